# Initial kernel scaffold; baseline (speedup 1.0000x reference)
import jax
import jax.numpy as jnp
from jax.experimental import pallas as pl


def kernel(x, W, router_scale):
    raise NotImplementedError("write your pallas kernel here")


# speedup vs baseline: 1.0859x; 1.0859x over previous
import functools

import jax
import jax.numpy as jnp
from jax.experimental import pallas as pl
from jax.experimental.pallas import tpu as pltpu

TOPK = 8
BLOCK_M = 1024


def _router_body(scale_ref, x_ref, wt_ref, w_out, i_out):
    logits = jnp.dot(x_ref[...], wt_ref[...], preferred_element_type=jnp.float32)
    logits = logits * scale_ref[0]
    n_exp = logits.shape[1]
    col = jax.lax.broadcasted_iota(jnp.int32, logits.shape, 1)
    vals, idxs = [], []
    cur = logits
    for _ in range(TOPK):
        m = jnp.max(cur, axis=1, keepdims=True)
        idx = jnp.min(jnp.where(cur == m, col, n_exp), axis=1, keepdims=True)
        vals.append(m)
        idxs.append(idx)
        cur = jnp.where(col == idx, -jnp.inf, cur)
    w = jnp.concatenate(vals, axis=1)
    e = jnp.exp(w - w[:, :1])
    w_out[...] = e / jnp.sum(e, axis=1, keepdims=True)
    i_out[...] = jnp.concatenate(idxs, axis=1)


@jax.jit
def kernel(x, W, router_scale):
    tokens, dim = x.shape
    n_exp = W.shape[0]
    wt = W.T
    grid = (tokens // BLOCK_M,)
    weights, indices = pl.pallas_call(
        _router_body,
        grid_spec=pltpu.PrefetchScalarGridSpec(
            num_scalar_prefetch=1,
            grid=grid,
            in_specs=[
                pl.BlockSpec((BLOCK_M, dim), lambda i, s: (i, 0)),
                pl.BlockSpec((dim, n_exp), lambda i, s: (0, 0)),
            ],
            out_specs=[
                pl.BlockSpec((BLOCK_M, TOPK), lambda i, s: (i, 0)),
                pl.BlockSpec((BLOCK_M, TOPK), lambda i, s: (i, 0)),
            ],
        ),
        out_shape=[
            jax.ShapeDtypeStruct((tokens, TOPK), jnp.float32),
            jax.ShapeDtypeStruct((tokens, TOPK), jnp.int32),
        ],
        compiler_params=pltpu.CompilerParams(
            dimension_semantics=("arbitrary",),
        ),
    )(router_scale, x, wt)
    return (weights, indices)


# speedup vs baseline: 1.4905x; 1.3726x over previous
import functools

import jax
import jax.numpy as jnp
from jax.experimental import pallas as pl
from jax.experimental.pallas import tpu as pltpu

TOPK = 8
BLOCK_M = 1024


def _router_body(scale_ref, x_ref, wt_ref, w_out, i_out):
    logits = jnp.dot(x_ref[...], wt_ref[...], preferred_element_type=jnp.float32)
    cur = logits.T * scale_ref[0]
    n_exp = cur.shape[0]
    row = jax.lax.broadcasted_iota(jnp.int32, cur.shape, 0)
    vals, idxs = [], []
    for _ in range(TOPK):
        m = jnp.max(cur, axis=0, keepdims=True)
        idx = jnp.min(jnp.where(cur == m, row, n_exp), axis=0, keepdims=True)
        vals.append(m)
        idxs.append(idx)
        cur = jnp.where(row == idx, -jnp.inf, cur)
    w = jnp.concatenate(vals, axis=0)
    e = jnp.exp(w - w[:1])
    w = e / jnp.sum(e, axis=0, keepdims=True)
    w_out[...] = w.T
    i_out[...] = jnp.concatenate(idxs, axis=0).T


@jax.jit
def kernel(x, W, router_scale):
    tokens, dim = x.shape
    n_exp = W.shape[0]
    wt = W.T
    grid = (tokens // BLOCK_M,)
    weights, indices = pl.pallas_call(
        _router_body,
        grid_spec=pltpu.PrefetchScalarGridSpec(
            num_scalar_prefetch=1,
            grid=grid,
            in_specs=[
                pl.BlockSpec((BLOCK_M, dim), lambda i, s: (i, 0)),
                pl.BlockSpec((dim, n_exp), lambda i, s: (0, 0)),
            ],
            out_specs=[
                pl.BlockSpec((BLOCK_M, TOPK), lambda i, s: (i, 0)),
                pl.BlockSpec((BLOCK_M, TOPK), lambda i, s: (i, 0)),
            ],
        ),
        out_shape=[
            jax.ShapeDtypeStruct((tokens, TOPK), jnp.float32),
            jax.ShapeDtypeStruct((tokens, TOPK), jnp.int32),
        ],
        compiler_params=pltpu.CompilerParams(
            dimension_semantics=("arbitrary",),
        ),
    )(router_scale, x, wt)
    return (weights, indices)
